# Initial kernel scaffold; baseline (speedup 1.0000x reference)
import jax
import jax.numpy as jnp
from jax.experimental import pallas as pl


def kernel(smiles_input, x, edge_index, batch, W_s1, b_s1, W_s2, b_s2, W_c1, b_c1, W_c2, b_c2, W_f, b_f, W_o, b_o):
    raise NotImplementedError("write your pallas kernel here")


# speedup vs baseline: 12.8787x; 12.8787x over previous
import functools

import jax
import jax.numpy as jnp
from jax import lax
from jax.experimental import pallas as pl
from jax.experimental.pallas import tpu as pltpu
from jax.experimental.pallas import tpu_sc as plsc

NC, NS = 2, 16
NW = NC * NS
CW = 16
EK = 80

def _sc_mesh():
    return plsc.VectorSubcoreMesh(
        core_axis_name="c", subcore_axis_name="s",
        num_cores=NC, num_subcores=NS)

_HIGH = lax.Precision.HIGHEST


CH = 200


def _sc_degree_counts(dst, n_nodes):
    e = dst.shape[0]
    per_w = e // NW
    n_chunks = per_w // EK
    n_ch = n_nodes // CH
    jmax = (n_ch + NS - 1) // NS

    @functools.partial(
        pl.kernel,
        out_type=jax.ShapeDtypeStruct((NC, n_nodes, CW), jnp.float32),
        mesh=_sc_mesh(),
        scratch_types=[
            pltpu.VMEM_SHARED((n_nodes, CW), jnp.float32),
            pltpu.VMEM((CH, CW), jnp.float32),
            pltpu.VMEM((EK, CW), jnp.float32),
            pltpu.VMEM((EK,), jnp.int32),
        ],
    )
    def k(dst_hbm, out_hbm, acc, zbuf, ones, didx):
        cid = lax.axis_index("c")
        sid = lax.axis_index("s")
        wid = sid * NC + cid
        zv = jnp.zeros((16,), jnp.float32)
        ov = jnp.ones((16,), jnp.float32)

        def fill_z(i, _):
            zbuf[i, pl.ds(0, 16)] = zv
            return 0

        lax.fori_loop(0, CH, fill_z, 0)

        def fill_o(i, _):
            ones[i, pl.ds(0, 16)] = ov
            return 0

        lax.fori_loop(0, EK, fill_o, 0)

        def zero_chunk(j, _):
            t = sid + j * NS

            @pl.when(t < n_ch)
            def _():
                pltpu.sync_copy(zbuf, acc.at[pl.ds(t * CH, CH)])

            return 0

        lax.fori_loop(0, jmax, zero_chunk, 0)
        plsc.subcore_barrier()

        base = wid * per_w

        def chunk(c, _):
            pltpu.sync_copy(dst_hbm.at[pl.ds(base + c * EK, EK)], didx)
            pltpu.sync_copy(ones, acc.at[didx], add=True)
            return 0

        lax.fori_loop(0, n_chunks, chunk, 0)
        plsc.subcore_barrier()

        def wb(j, _):
            t = sid + j * NS

            @pl.when(t < n_ch)
            def _():
                sl = pl.ds(t * CH, CH)
                pltpu.sync_copy(acc.at[sl], zbuf)
                pltpu.sync_copy(zbuf, out_hbm.at[cid, sl])

            return 0

        lax.fori_loop(0, jmax, wb, 0)

    return k(dst)


def _sc_edge_agg(p, src, dst):
    n, d = p.shape
    e = src.shape[0]
    per_w = e // NW
    n_chunks = per_w // EK
    n_ch = n // CH
    jmax = (n_ch + NS - 1) // NS

    @functools.partial(
        pl.kernel,
        out_type=jax.ShapeDtypeStruct((NC, n, d), jnp.float32),
        mesh=_sc_mesh(),
        scratch_types=[
            pltpu.VMEM_SHARED((n, d), jnp.float32),
            pltpu.VMEM((CH, d), jnp.float32),
            pltpu.VMEM((EK,), jnp.int32),
            pltpu.VMEM((EK,), jnp.int32),
            pltpu.VMEM((EK, d), jnp.float32),
            pltpu.SemaphoreType.DMA,
        ],
    )
    def k(p_hbm, src_hbm, dst_hbm, out_hbm, acc, zbuf, sidx, didx, rows, gsem):
        cid = lax.axis_index("c")
        sid = lax.axis_index("s")
        wid = sid * NC + cid
        zv = jnp.zeros((16,), jnp.float32)

        def fill_z(i, _):
            def col(j, _):
                zbuf[i, pl.ds(j * 16, 16)] = zv
                return 0

            lax.fori_loop(0, d // 16, col, 0)
            return 0

        lax.fori_loop(0, CH, fill_z, 0)

        def zero_chunk(j, _):
            t = sid + j * NS

            @pl.when(t < n_ch)
            def _():
                pltpu.sync_copy(zbuf, acc.at[pl.ds(t * CH, CH)])

            return 0

        lax.fori_loop(0, jmax, zero_chunk, 0)
        plsc.subcore_barrier()

        base = wid * per_w

        def chunk(c, _):
            off = base + c * EK
            pltpu.sync_copy(src_hbm.at[pl.ds(off, EK)], sidx)
            pltpu.sync_copy(dst_hbm.at[pl.ds(off, EK)], didx)
            pltpu.async_copy(p_hbm.at[sidx], rows, gsem).wait()
            pltpu.sync_copy(rows, acc.at[didx], add=True)
            return 0

        lax.fori_loop(0, n_chunks, chunk, 0)
        plsc.subcore_barrier()

        def wb(j, _):
            t = sid + j * NS

            @pl.when(t < n_ch)
            def _():
                sl = pl.ds(t * CH, CH)
                pltpu.sync_copy(acc.at[sl], zbuf)
                pltpu.sync_copy(zbuf, out_hbm.at[cid, sl])

            return 0

        lax.fori_loop(0, jmax, wb, 0)

    return k(p, src, dst)


def _tc_prep(x, W_c1, cnt, smiles, W_s1, b_s1, W_s2, b_s2):
    n, d = x.shape
    bn = 1000
    nb = n // bn
    bs, ss = smiles.shape
    h = W_s1.shape[1]

    def body(x_ref, w_ref, cnt_ref, sm_ref, ws1_ref, bs1_ref, ws2_ref,
             bs2_ref, p_ref, s_ref):
        i = pl.program_id(0)
        deg = cnt_ref[0, :, 0:1] + cnt_ref[1, :, 0:1] + 1.0
        dinv = lax.rsqrt(deg)
        p_ref[...] = jnp.dot(x_ref[...], w_ref[...],
                             preferred_element_type=jnp.float32,
                             precision=_HIGH) * dinv

        @pl.when(i == 0)
        def _():
            t = jnp.maximum(
                jnp.dot(sm_ref[...], ws1_ref[...],
                        preferred_element_type=jnp.float32,
                        precision=_HIGH) + bs1_ref[...], 0.0)
            s_ref[...] = jnp.dot(t, ws2_ref[...],
                                 preferred_element_type=jnp.float32,
                                 precision=_HIGH) + bs2_ref[...]

    return pl.pallas_call(
        body,
        grid=(nb,),
        in_specs=[
            pl.BlockSpec((bn, d), lambda i: (i, 0)),
            pl.BlockSpec((d, d), lambda i: (0, 0)),
            pl.BlockSpec((NC, bn, CW), lambda i: (0, i, 0)),
            pl.BlockSpec((bs, ss), lambda i: (0, 0)),
            pl.BlockSpec((ss, h), lambda i: (0, 0)),
            pl.BlockSpec((1, h), lambda i: (0, 0)),
            pl.BlockSpec((h, h), lambda i: (0, 0)),
            pl.BlockSpec((1, h), lambda i: (0, 0)),
        ],
        out_specs=[
            pl.BlockSpec((bn, d), lambda i: (i, 0)),
            pl.BlockSpec((bs, h), lambda i: (0, 0)),
        ],
        out_shape=[
            jax.ShapeDtypeStruct((n, d), jnp.float32),
            jax.ShapeDtypeStruct((bs, h), jnp.float32),
        ],
    )(x, W_c1, cnt, smiles, W_s1, b_s1, W_s2, b_s2)


def _tc_mid(agg, p1, cnt, b_c1, W_c2):
    _, n, d = agg.shape
    bn = 1000
    nb = n // bn

    def body(agg_ref, p_ref, cnt_ref, b_ref, w_ref, o_ref):
        deg = cnt_ref[0, :, 0:1] + cnt_ref[1, :, 0:1] + 1.0
        dinv = lax.rsqrt(deg)
        h1 = jnp.maximum(
            dinv * (agg_ref[0] + agg_ref[1] + p_ref[...]) + b_ref[...], 0.0)
        o_ref[...] = jnp.dot(h1, w_ref[...],
                             preferred_element_type=jnp.float32,
                             precision=_HIGH) * dinv

    return pl.pallas_call(
        body,
        grid=(nb,),
        in_specs=[
            pl.BlockSpec((NC, bn, d), lambda i: (0, i, 0)),
            pl.BlockSpec((bn, d), lambda i: (i, 0)),
            pl.BlockSpec((NC, bn, CW), lambda i: (0, i, 0)),
            pl.BlockSpec((1, d), lambda i: (0, 0)),
            pl.BlockSpec((d, d), lambda i: (0, 0)),
        ],
        out_specs=pl.BlockSpec((bn, d), lambda i: (i, 0)),
        out_shape=jax.ShapeDtypeStruct((n, d), jnp.float32),
    )(agg, p1, cnt, b_c1, W_c2)


def _tc_final(agg, p2, cnt, b_c2, batch3, s, W_fs, W_fg, b_f, W_o_pad,
              b_o_pad):
    _, n, d = agg.shape
    bn = 1000
    nb = n // bn
    bs = s.shape[0]

    def body(agg_ref, p_ref, cnt_ref, b_ref, bat_ref, s_ref, wfs_ref,
             wfg_ref, bf_ref, wo_ref, bo_ref, o_ref, gsum, cntv):
        i = pl.program_id(0)

        @pl.when(i == 0)
        def _():
            gsum[...] = jnp.zeros_like(gsum)
            cntv[...] = jnp.zeros_like(cntv)

        deg = cnt_ref[0, :, 0:1] + cnt_ref[1, :, 0:1] + 1.0
        dinv = lax.rsqrt(deg)
        h2 = jnp.maximum(
            dinv * (agg_ref[0] + agg_ref[1] + p_ref[...]) + b_ref[...], 0.0)
        bvals = bat_ref[0]
        biota = lax.broadcasted_iota(jnp.int32, (bs, bn), 0)
        mask = (biota == bvals).astype(jnp.float32)
        gsum[...] += jnp.dot(mask, h2, preferred_element_type=jnp.float32,
                             precision=_HIGH)
        cntv[...] += jnp.sum(mask, axis=1, keepdims=True)

        @pl.when(i == nb - 1)
        def _():
            g = gsum[...] / jnp.maximum(cntv[...], 1.0)
            fused = jnp.maximum(
                jnp.dot(s_ref[...], wfs_ref[...],
                        preferred_element_type=jnp.float32, precision=_HIGH)
                + jnp.dot(g, wfg_ref[...],
                          preferred_element_type=jnp.float32, precision=_HIGH)
                + bf_ref[...], 0.0)
            o_ref[...] = jnp.dot(fused, wo_ref[...],
                                 preferred_element_type=jnp.float32,
                                 precision=_HIGH) + bo_ref[...]

    h = W_fs.shape[1]
    return pl.pallas_call(
        body,
        grid=(nb,),
        in_specs=[
            pl.BlockSpec((NC, bn, d), lambda i: (0, i, 0)),
            pl.BlockSpec((bn, d), lambda i: (i, 0)),
            pl.BlockSpec((NC, bn, CW), lambda i: (0, i, 0)),
            pl.BlockSpec((1, d), lambda i: (0, 0)),
            pl.BlockSpec((1, 1, bn), lambda i: (i, 0, 0)),
            pl.BlockSpec((bs, h), lambda i: (0, 0)),
            pl.BlockSpec((h, h), lambda i: (0, 0)),
            pl.BlockSpec((h, h), lambda i: (0, 0)),
            pl.BlockSpec((1, h), lambda i: (0, 0)),
            pl.BlockSpec((h, h), lambda i: (0, 0)),
            pl.BlockSpec((1, h), lambda i: (0, 0)),
        ],
        out_specs=pl.BlockSpec((bs, h), lambda i: (0, 0)),
        out_shape=jax.ShapeDtypeStruct((bs, h), jnp.float32),
        scratch_shapes=[
            pltpu.VMEM((bs, h), jnp.float32),
            pltpu.VMEM((bs, 1), jnp.float32),
        ],
    )(agg, p2, cnt, b_c2, batch3, s, W_fs, W_fg, b_f, W_o_pad, b_o_pad)


def kernel(smiles_input, x, edge_index, batch, W_s1, b_s1, W_s2, b_s2,
           W_c1, b_c1, W_c2, b_c2, W_f, b_f, W_o, b_o):
    n, d = x.shape
    src = edge_index[0]
    dst = edge_index[1]
    h = W_s1.shape[1]
    c_out = W_o.shape[1]

    cnt = _sc_degree_counts(dst, n)
    p1, s = _tc_prep(x, W_c1, cnt, smiles_input,
                     W_s1, b_s1.reshape(1, -1), W_s2, b_s2.reshape(1, -1))
    agg1 = _sc_edge_agg(p1, src, dst)
    p2 = _tc_mid(agg1, p1, cnt, b_c1.reshape(1, -1), W_c2)
    agg2 = _sc_edge_agg(p2, src, dst)

    bn = 1000
    batch3 = batch.reshape(n // bn, 1, bn)
    W_o_pad = jnp.pad(W_o, ((0, 0), (0, h - c_out)))
    b_o_pad = jnp.pad(b_o, (0, h - c_out)).reshape(1, -1)
    outp = _tc_final(agg2, p2, cnt, b_c2.reshape(1, -1), batch3, s,
                     W_f[:h], W_f[h:], b_f.reshape(1, -1), W_o_pad, b_o_pad)
    return outp[:, :c_out]


# speedup vs baseline: 16.7343x; 1.2994x over previous
import functools

import jax
import jax.numpy as jnp
from jax import lax
from jax.experimental import pallas as pl
from jax.experimental.pallas import tpu as pltpu
from jax.experimental.pallas import tpu_sc as plsc

NC, NS = 2, 16
NW = NC * NS
CW = 16
EK = 80

def _sc_mesh():
    return plsc.VectorSubcoreMesh(
        core_axis_name="c", subcore_axis_name="s",
        num_cores=NC, num_subcores=NS)

_HIGH = lax.Precision.HIGHEST


CH = 200


def _sc_degree_counts(dst, n_nodes):
    e = dst.shape[0]
    per_w = e // NW
    nb = per_w // EK
    half = nb // 2
    n_ch = n_nodes // CH
    jmax = (n_ch + NS - 1) // NS

    @functools.partial(
        pl.kernel,
        out_type=jax.ShapeDtypeStruct((NC, n_nodes, CW), jnp.float32),
        mesh=_sc_mesh(),
        scratch_types=[
            pltpu.VMEM_SHARED((n_nodes, CW), jnp.float32),
            pltpu.VMEM((CH, CW), jnp.float32),
            pltpu.VMEM((EK, CW), jnp.float32),
            pltpu.VMEM((EK,), jnp.int32),
            pltpu.VMEM((EK,), jnp.int32),
            pltpu.SemaphoreType.DMA,
            pltpu.SemaphoreType.DMA,
        ],
    )
    def k(dst_hbm, out_hbm, acc, zbuf, ones, didx0, didx1, dsem0, dsem1):
        cid = lax.axis_index("c")
        sid = lax.axis_index("s")
        wid = sid * NC + cid
        zv = jnp.zeros((16,), jnp.float32)
        ov = jnp.ones((16,), jnp.float32)

        def fill_z(i, _):
            zbuf[i, pl.ds(0, 16)] = zv
            return 0

        lax.fori_loop(0, CH, fill_z, 0)

        def fill_o(i, _):
            ones[i, pl.ds(0, 16)] = ov
            return 0

        lax.fori_loop(0, EK, fill_o, 0)

        def zero_chunk(j, _):
            t = sid + j * NS

            @pl.when(t < n_ch)
            def _():
                pltpu.sync_copy(zbuf, acc.at[pl.ds(t * CH, CH)])

            return 0

        lax.fori_loop(0, jmax, zero_chunk, 0)
        plsc.subcore_barrier()

        base = wid * per_w

        def chunk(c, _):
            pltpu.sync_copy(dst_hbm.at[pl.ds(base + c * EK, EK)], didx0)
            pltpu.sync_copy(ones, acc.at[didx0], add=True)
            return 0

        lax.fori_loop(0, nb, chunk, 0)
        plsc.subcore_barrier()

        def wb(j, _):
            t = sid + j * NS

            @pl.when(t < n_ch)
            def _():
                sl = pl.ds(t * CH, CH)
                pltpu.sync_copy(acc.at[sl], zbuf)
                pltpu.sync_copy(zbuf, out_hbm.at[cid, sl])

            return 0

        lax.fori_loop(0, jmax, wb, 0)

    return k(dst)


def _sc_edge_agg(p, src, dst):
    n, d = p.shape
    e = src.shape[0]
    per_w = e // NW
    nb = per_w // EK
    half = nb // 2
    n_zc = n // EK
    jmax = (n_zc + NS - 1) // NS

    @functools.partial(
        pl.kernel,
        out_type=jax.ShapeDtypeStruct((NC, n, d), jnp.float32),
        mesh=_sc_mesh(),
        scratch_types=[
            pltpu.VMEM_SHARED((n, d), jnp.float32),
            pltpu.VMEM((EK,), jnp.int32),
            pltpu.VMEM((EK,), jnp.int32),
            pltpu.VMEM((EK,), jnp.int32),
            pltpu.VMEM((EK,), jnp.int32),
            pltpu.VMEM((EK, d), jnp.float32),
            pltpu.VMEM((EK, d), jnp.float32),
            pltpu.SemaphoreType.DMA,
            pltpu.SemaphoreType.DMA,
            pltpu.SemaphoreType.DMA,
            pltpu.SemaphoreType.DMA,
            pltpu.SemaphoreType.DMA,
            pltpu.SemaphoreType.DMA,
        ],
    )
    def k(p_hbm, src_hbm, dst_hbm, out_hbm, acc, sidx0, sidx1, didx0, didx1,
          rows0, rows1, g0, g1, si0, si1, di0, di1):
        cid = lax.axis_index("c")
        sid = lax.axis_index("s")
        wid = sid * NC + cid
        zv = jnp.zeros((16,), jnp.float32)

        def fill_z(i, _):
            def col(j, _):
                rows0[i, pl.ds(j * 16, 16)] = zv
                return 0

            lax.fori_loop(0, d // 16, col, 0)
            return 0

        lax.fori_loop(0, EK, fill_z, 0)

        def zero_chunk(j, _):
            t = sid + j * NS

            @pl.when(t < n_zc)
            def _():
                pltpu.sync_copy(rows0, acc.at[pl.ds(t * EK, EK)])

            return 0

        lax.fori_loop(0, jmax, zero_chunk, 0)
        plsc.subcore_barrier()

        base = wid * per_w

        def ssl(c):
            return src_hbm.at[pl.ds(base + c * EK, EK)]

        def dsl(c):
            return dst_hbm.at[pl.ds(base + c * EK, EK)]

        def pair(cc, _):
            c0 = 2 * cc
            pltpu.sync_copy(ssl(c0), sidx0)
            pltpu.sync_copy(dsl(c0), didx0)
            d0 = pltpu.async_copy(p_hbm.at[sidx0], rows0, g0)
            pltpu.sync_copy(ssl(c0 + 1), sidx1)
            pltpu.sync_copy(dsl(c0 + 1), didx1)
            d1 = pltpu.async_copy(p_hbm.at[sidx1], rows1, g1)
            d0.wait()
            pltpu.sync_copy(rows0, acc.at[didx0], add=True)
            d1.wait()
            pltpu.sync_copy(rows1, acc.at[didx1], add=True)
            return 0

        lax.fori_loop(0, half, pair, 0)
        pltpu.sync_copy(ssl(nb - 1), sidx0)
        pltpu.sync_copy(dsl(nb - 1), didx0)
        pltpu.async_copy(p_hbm.at[sidx0], rows0, g0).wait()
        pltpu.sync_copy(rows0, acc.at[didx0], add=True)
        plsc.subcore_barrier()

        def wb(j, _):
            t = sid + j * NS

            @pl.when(t < n_zc)
            def _():
                sl = pl.ds(t * EK, EK)
                pltpu.sync_copy(acc.at[sl], rows0)
                pltpu.sync_copy(rows0, out_hbm.at[cid, sl])

            return 0

        lax.fori_loop(0, jmax, wb, 0)

    return k(p, src, dst)


def _tc_prep(x, W_c1, cnt, smiles, W_s1, b_s1, W_s2, b_s2):
    n, d = x.shape
    bn = 1000
    nb = n // bn
    bs, ss = smiles.shape
    h = W_s1.shape[1]

    def body(x_ref, w_ref, cnt_ref, sm_ref, ws1_ref, bs1_ref, ws2_ref,
             bs2_ref, p_ref, s_ref):
        i = pl.program_id(0)
        deg = cnt_ref[0, :, 0:1] + cnt_ref[1, :, 0:1] + 1.0
        dinv = lax.rsqrt(deg)
        p_ref[...] = jnp.dot(x_ref[...], w_ref[...],
                             preferred_element_type=jnp.float32,
                             precision=_HIGH) * dinv

        @pl.when(i == 0)
        def _():
            t = jnp.maximum(
                jnp.dot(sm_ref[...], ws1_ref[...],
                        preferred_element_type=jnp.float32,
                        precision=_HIGH) + bs1_ref[...], 0.0)
            s_ref[...] = jnp.dot(t, ws2_ref[...],
                                 preferred_element_type=jnp.float32,
                                 precision=_HIGH) + bs2_ref[...]

    return pl.pallas_call(
        body,
        grid=(nb,),
        in_specs=[
            pl.BlockSpec((bn, d), lambda i: (i, 0)),
            pl.BlockSpec((d, d), lambda i: (0, 0)),
            pl.BlockSpec((NC, bn, CW), lambda i: (0, i, 0)),
            pl.BlockSpec((bs, ss), lambda i: (0, 0)),
            pl.BlockSpec((ss, h), lambda i: (0, 0)),
            pl.BlockSpec((1, h), lambda i: (0, 0)),
            pl.BlockSpec((h, h), lambda i: (0, 0)),
            pl.BlockSpec((1, h), lambda i: (0, 0)),
        ],
        out_specs=[
            pl.BlockSpec((bn, d), lambda i: (i, 0)),
            pl.BlockSpec((bs, h), lambda i: (0, 0)),
        ],
        out_shape=[
            jax.ShapeDtypeStruct((n, d), jnp.float32),
            jax.ShapeDtypeStruct((bs, h), jnp.float32),
        ],
    )(x, W_c1, cnt, smiles, W_s1, b_s1, W_s2, b_s2)


def _tc_mid(agg, p1, cnt, b_c1, W_c2):
    _, n, d = agg.shape
    bn = 1000
    nb = n // bn

    def body(agg_ref, p_ref, cnt_ref, b_ref, w_ref, o_ref):
        deg = cnt_ref[0, :, 0:1] + cnt_ref[1, :, 0:1] + 1.0
        dinv = lax.rsqrt(deg)
        h1 = jnp.maximum(
            dinv * (agg_ref[0] + agg_ref[1] + p_ref[...]) + b_ref[...], 0.0)
        o_ref[...] = jnp.dot(h1, w_ref[...],
                             preferred_element_type=jnp.float32,
                             precision=_HIGH) * dinv

    return pl.pallas_call(
        body,
        grid=(nb,),
        in_specs=[
            pl.BlockSpec((NC, bn, d), lambda i: (0, i, 0)),
            pl.BlockSpec((bn, d), lambda i: (i, 0)),
            pl.BlockSpec((NC, bn, CW), lambda i: (0, i, 0)),
            pl.BlockSpec((1, d), lambda i: (0, 0)),
            pl.BlockSpec((d, d), lambda i: (0, 0)),
        ],
        out_specs=pl.BlockSpec((bn, d), lambda i: (i, 0)),
        out_shape=jax.ShapeDtypeStruct((n, d), jnp.float32),
    )(agg, p1, cnt, b_c1, W_c2)


def _tc_final(agg, p2, cnt, b_c2, batch3, s, W_fs, W_fg, b_f, W_o_pad,
              b_o_pad):
    _, n, d = agg.shape
    bn = 1000
    nb = n // bn
    bs = s.shape[0]

    def body(agg_ref, p_ref, cnt_ref, b_ref, bat_ref, s_ref, wfs_ref,
             wfg_ref, bf_ref, wo_ref, bo_ref, o_ref, gsum, cntv):
        i = pl.program_id(0)

        @pl.when(i == 0)
        def _():
            gsum[...] = jnp.zeros_like(gsum)
            cntv[...] = jnp.zeros_like(cntv)

        deg = cnt_ref[0, :, 0:1] + cnt_ref[1, :, 0:1] + 1.0
        dinv = lax.rsqrt(deg)
        h2 = jnp.maximum(
            dinv * (agg_ref[0] + agg_ref[1] + p_ref[...]) + b_ref[...], 0.0)
        bvals = bat_ref[0]
        biota = lax.broadcasted_iota(jnp.int32, (bs, bn), 0)
        mask = (biota == bvals).astype(jnp.float32)
        gsum[...] += jnp.dot(mask, h2, preferred_element_type=jnp.float32,
                             precision=_HIGH)
        cntv[...] += jnp.sum(mask, axis=1, keepdims=True)

        @pl.when(i == nb - 1)
        def _():
            g = gsum[...] / jnp.maximum(cntv[...], 1.0)
            fused = jnp.maximum(
                jnp.dot(s_ref[...], wfs_ref[...],
                        preferred_element_type=jnp.float32, precision=_HIGH)
                + jnp.dot(g, wfg_ref[...],
                          preferred_element_type=jnp.float32, precision=_HIGH)
                + bf_ref[...], 0.0)
            o_ref[...] = jnp.dot(fused, wo_ref[...],
                                 preferred_element_type=jnp.float32,
                                 precision=_HIGH) + bo_ref[...]

    h = W_fs.shape[1]
    return pl.pallas_call(
        body,
        grid=(nb,),
        in_specs=[
            pl.BlockSpec((NC, bn, d), lambda i: (0, i, 0)),
            pl.BlockSpec((bn, d), lambda i: (i, 0)),
            pl.BlockSpec((NC, bn, CW), lambda i: (0, i, 0)),
            pl.BlockSpec((1, d), lambda i: (0, 0)),
            pl.BlockSpec((1, 1, bn), lambda i: (i, 0, 0)),
            pl.BlockSpec((bs, h), lambda i: (0, 0)),
            pl.BlockSpec((h, h), lambda i: (0, 0)),
            pl.BlockSpec((h, h), lambda i: (0, 0)),
            pl.BlockSpec((1, h), lambda i: (0, 0)),
            pl.BlockSpec((h, h), lambda i: (0, 0)),
            pl.BlockSpec((1, h), lambda i: (0, 0)),
        ],
        out_specs=pl.BlockSpec((bs, h), lambda i: (0, 0)),
        out_shape=jax.ShapeDtypeStruct((bs, h), jnp.float32),
        scratch_shapes=[
            pltpu.VMEM((bs, h), jnp.float32),
            pltpu.VMEM((bs, 1), jnp.float32),
        ],
    )(agg, p2, cnt, b_c2, batch3, s, W_fs, W_fg, b_f, W_o_pad, b_o_pad)


def kernel(smiles_input, x, edge_index, batch, W_s1, b_s1, W_s2, b_s2,
           W_c1, b_c1, W_c2, b_c2, W_f, b_f, W_o, b_o):
    n, d = x.shape
    src = edge_index[0]
    dst = edge_index[1]
    h = W_s1.shape[1]
    c_out = W_o.shape[1]

    cnt = _sc_degree_counts(dst, n)
    p1, s = _tc_prep(x, W_c1, cnt, smiles_input,
                     W_s1, b_s1.reshape(1, -1), W_s2, b_s2.reshape(1, -1))
    agg1 = _sc_edge_agg(p1, src, dst)
    p2 = _tc_mid(agg1, p1, cnt, b_c1.reshape(1, -1), W_c2)
    agg2 = _sc_edge_agg(p2, src, dst)

    bn = 1000
    batch3 = batch.reshape(n // bn, 1, bn)
    W_o_pad = jnp.pad(W_o, ((0, 0), (0, h - c_out)))
    b_o_pad = jnp.pad(b_o, (0, h - c_out)).reshape(1, -1)
    outp = _tc_final(agg2, p2, cnt, b_c2.reshape(1, -1), batch3, s,
                     W_f[:h], W_f[h:], b_f.reshape(1, -1), W_o_pad, b_o_pad)
    return outp[:, :c_out]


# speedup vs baseline: 23.8970x; 1.4280x over previous
import functools

import jax
import jax.numpy as jnp
from jax import lax
from jax.experimental import pallas as pl
from jax.experimental.pallas import tpu as pltpu
from jax.experimental.pallas import tpu_sc as plsc

NC, NS = 2, 16
NW = NC * NS
CW = 16
EK = 80

def _sc_mesh():
    return plsc.VectorSubcoreMesh(
        core_axis_name="c", subcore_axis_name="s",
        num_cores=NC, num_subcores=NS)

_HIGH = lax.Precision.HIGHEST


CH = 200


def _sc_degree_counts(dst, n_nodes):
    e = dst.shape[0]
    per_w = e // NW
    nb = per_w // EK
    half = nb // 2
    n_ch = n_nodes // CH
    jmax = (n_ch + NS - 1) // NS

    @functools.partial(
        pl.kernel,
        out_type=jax.ShapeDtypeStruct((NC, n_nodes, CW), jnp.float32),
        mesh=_sc_mesh(),
        scratch_types=[
            pltpu.VMEM_SHARED((n_nodes, CW), jnp.float32),
            pltpu.VMEM((CH, CW), jnp.float32),
            pltpu.VMEM((EK, CW), jnp.float32),
            pltpu.VMEM((EK,), jnp.int32),
            pltpu.VMEM((EK,), jnp.int32),
            pltpu.SemaphoreType.DMA,
            pltpu.SemaphoreType.DMA,
        ],
    )
    def k(dst_hbm, out_hbm, acc, zbuf, ones, didx0, didx1, dsem0, dsem1):
        cid = lax.axis_index("c")
        sid = lax.axis_index("s")
        wid = sid * NC + cid
        zv = jnp.zeros((16,), jnp.float32)
        ov = jnp.ones((16,), jnp.float32)

        def fill_z(i, _):
            zbuf[i, pl.ds(0, 16)] = zv
            return 0

        lax.fori_loop(0, CH, fill_z, 0)

        def fill_o(i, _):
            ones[i, pl.ds(0, 16)] = ov
            return 0

        lax.fori_loop(0, EK, fill_o, 0)

        def zero_chunk(j, _):
            t = sid + j * NS

            @pl.when(t < n_ch)
            def _():
                pltpu.sync_copy(zbuf, acc.at[pl.ds(t * CH, CH)])

            return 0

        lax.fori_loop(0, jmax, zero_chunk, 0)
        plsc.subcore_barrier()

        base = wid * per_w

        def dsl(c):
            return dst_hbm.at[pl.ds(base + c * EK, EK)]

        def idrain(buf, sem):
            pltpu.make_async_copy(dst_hbm.at[pl.ds(0, EK)], buf, sem).wait()

        pltpu.async_copy(dsl(0), didx0, dsem0)
        pltpu.async_copy(dsl(1), didx1, dsem1)

        def pair(cc, _):
            c0 = 2 * cc
            idrain(didx0, dsem0)
            pltpu.sync_copy(ones, acc.at[didx0], add=True)
            pltpu.async_copy(dsl(c0 + 2), didx0, dsem0)
            idrain(didx1, dsem1)
            pltpu.sync_copy(ones, acc.at[didx1], add=True)

            @pl.when(cc < half - 1)
            def _():
                pltpu.async_copy(dsl(c0 + 3), didx1, dsem1)

            return 0

        lax.fori_loop(0, half, pair, 0)
        idrain(didx0, dsem0)
        pltpu.sync_copy(ones, acc.at[didx0], add=True)
        plsc.subcore_barrier()

        def wb(j, _):
            t = sid + j * NS

            @pl.when(t < n_ch)
            def _():
                sl = pl.ds(t * CH, CH)
                pltpu.sync_copy(acc.at[sl], zbuf)
                pltpu.sync_copy(zbuf, out_hbm.at[cid, sl])

            return 0

        lax.fori_loop(0, jmax, wb, 0)

    return k(dst)


def _sc_edge_agg(p, src, dst):
    n, d = p.shape
    e = src.shape[0]
    per_w = e // NW
    nb = per_w // EK
    half = nb // 2
    n_zc = n // EK
    jmax = (n_zc + NS - 1) // NS

    @functools.partial(
        pl.kernel,
        out_type=jax.ShapeDtypeStruct((NC, n, d), jnp.float32),
        mesh=_sc_mesh(),
        scratch_types=[
            pltpu.VMEM_SHARED((n, d), jnp.float32),
            pltpu.VMEM((EK,), jnp.int32),
            pltpu.VMEM((EK,), jnp.int32),
            pltpu.VMEM((EK,), jnp.int32),
            pltpu.VMEM((EK,), jnp.int32),
            pltpu.VMEM((EK, d), jnp.float32),
            pltpu.VMEM((EK, d), jnp.float32),
            pltpu.SemaphoreType.DMA,
            pltpu.SemaphoreType.DMA,
            pltpu.SemaphoreType.DMA,
            pltpu.SemaphoreType.DMA,
            pltpu.SemaphoreType.DMA,
            pltpu.SemaphoreType.DMA,
        ],
    )
    def k(p_hbm, src_hbm, dst_hbm, out_hbm, acc, sidx0, sidx1, didx0, didx1,
          rows0, rows1, g0, g1, si0, si1, di0, di1):
        cid = lax.axis_index("c")
        sid = lax.axis_index("s")
        wid = sid * NC + cid
        zv = jnp.zeros((16,), jnp.float32)

        def fill_z(i, _):
            def col(j, _):
                rows0[i, pl.ds(j * 16, 16)] = zv
                return 0

            lax.fori_loop(0, d // 16, col, 0)
            return 0

        lax.fori_loop(0, EK, fill_z, 0)

        def zero_chunk(j, _):
            t = sid + j * NS

            @pl.when(t < n_zc)
            def _():
                pltpu.sync_copy(rows0, acc.at[pl.ds(t * EK, EK)])

            return 0

        lax.fori_loop(0, jmax, zero_chunk, 0)
        plsc.subcore_barrier()

        base = wid * per_w

        def ssl(c):
            return src_hbm.at[pl.ds(base + c * EK, EK)]

        def dsl(c):
            return dst_hbm.at[pl.ds(base + c * EK, EK)]

        def idrain(buf, sem):
            pltpu.make_async_copy(src_hbm.at[pl.ds(0, EK)], buf, sem).wait()

        pltpu.async_copy(ssl(0), sidx0, si0)
        pltpu.async_copy(dsl(0), didx0, di0)
        pltpu.async_copy(ssl(1), sidx1, si1)
        pltpu.async_copy(dsl(1), didx1, di1)

        def pair(cc, _):
            c0 = 2 * cc
            idrain(sidx0, si0)
            d0 = pltpu.async_copy(p_hbm.at[sidx0], rows0, g0)
            idrain(sidx1, si1)
            d1 = pltpu.async_copy(p_hbm.at[sidx1], rows1, g1)
            d0.wait()
            idrain(didx0, di0)
            pltpu.sync_copy(rows0, acc.at[didx0], add=True)
            pltpu.async_copy(ssl(c0 + 2), sidx0, si0)
            pltpu.async_copy(dsl(c0 + 2), didx0, di0)
            d1.wait()
            idrain(didx1, di1)
            pltpu.sync_copy(rows1, acc.at[didx1], add=True)

            @pl.when(cc < half - 1)
            def _():
                pltpu.async_copy(ssl(c0 + 3), sidx1, si1)
                pltpu.async_copy(dsl(c0 + 3), didx1, di1)

            return 0

        lax.fori_loop(0, half, pair, 0)
        idrain(sidx0, si0)
        pltpu.async_copy(p_hbm.at[sidx0], rows0, g0).wait()
        idrain(didx0, di0)
        pltpu.sync_copy(rows0, acc.at[didx0], add=True)
        plsc.subcore_barrier()

        def wb(j, _):
            t = sid + j * NS

            @pl.when(t < n_zc)
            def _():
                sl = pl.ds(t * EK, EK)
                pltpu.sync_copy(acc.at[sl], rows0)
                pltpu.sync_copy(rows0, out_hbm.at[cid, sl])

            return 0

        lax.fori_loop(0, jmax, wb, 0)

    return k(p, src, dst)


def _tc_prep(x, W_c1, cnt, smiles, W_s1, b_s1, W_s2, b_s2):
    n, d = x.shape
    bn = 1000
    nb = n // bn
    bs, ss = smiles.shape
    h = W_s1.shape[1]

    def body(x_ref, w_ref, cnt_ref, sm_ref, ws1_ref, bs1_ref, ws2_ref,
             bs2_ref, p_ref, s_ref):
        i = pl.program_id(0)
        deg = cnt_ref[0, :, 0:1] + cnt_ref[1, :, 0:1] + 1.0
        dinv = lax.rsqrt(deg)
        p_ref[...] = jnp.dot(x_ref[...], w_ref[...],
                             preferred_element_type=jnp.float32,
                             precision=_HIGH) * dinv

        @pl.when(i == 0)
        def _():
            t = jnp.maximum(
                jnp.dot(sm_ref[...], ws1_ref[...],
                        preferred_element_type=jnp.float32,
                        precision=_HIGH) + bs1_ref[...], 0.0)
            s_ref[...] = jnp.dot(t, ws2_ref[...],
                                 preferred_element_type=jnp.float32,
                                 precision=_HIGH) + bs2_ref[...]

    return pl.pallas_call(
        body,
        grid=(nb,),
        in_specs=[
            pl.BlockSpec((bn, d), lambda i: (i, 0)),
            pl.BlockSpec((d, d), lambda i: (0, 0)),
            pl.BlockSpec((NC, bn, CW), lambda i: (0, i, 0)),
            pl.BlockSpec((bs, ss), lambda i: (0, 0)),
            pl.BlockSpec((ss, h), lambda i: (0, 0)),
            pl.BlockSpec((1, h), lambda i: (0, 0)),
            pl.BlockSpec((h, h), lambda i: (0, 0)),
            pl.BlockSpec((1, h), lambda i: (0, 0)),
        ],
        out_specs=[
            pl.BlockSpec((bn, d), lambda i: (i, 0)),
            pl.BlockSpec((bs, h), lambda i: (0, 0)),
        ],
        out_shape=[
            jax.ShapeDtypeStruct((n, d), jnp.float32),
            jax.ShapeDtypeStruct((bs, h), jnp.float32),
        ],
    )(x, W_c1, cnt, smiles, W_s1, b_s1, W_s2, b_s2)


def _tc_mid(agg, p1, cnt, b_c1, W_c2):
    _, n, d = agg.shape
    bn = 1000
    nb = n // bn

    def body(agg_ref, p_ref, cnt_ref, b_ref, w_ref, o_ref):
        deg = cnt_ref[0, :, 0:1] + cnt_ref[1, :, 0:1] + 1.0
        dinv = lax.rsqrt(deg)
        h1 = jnp.maximum(
            dinv * (agg_ref[0] + agg_ref[1] + p_ref[...]) + b_ref[...], 0.0)
        o_ref[...] = jnp.dot(h1, w_ref[...],
                             preferred_element_type=jnp.float32,
                             precision=_HIGH) * dinv

    return pl.pallas_call(
        body,
        grid=(nb,),
        in_specs=[
            pl.BlockSpec((NC, bn, d), lambda i: (0, i, 0)),
            pl.BlockSpec((bn, d), lambda i: (i, 0)),
            pl.BlockSpec((NC, bn, CW), lambda i: (0, i, 0)),
            pl.BlockSpec((1, d), lambda i: (0, 0)),
            pl.BlockSpec((d, d), lambda i: (0, 0)),
        ],
        out_specs=pl.BlockSpec((bn, d), lambda i: (i, 0)),
        out_shape=jax.ShapeDtypeStruct((n, d), jnp.float32),
    )(agg, p1, cnt, b_c1, W_c2)


def _tc_final(agg, p2, cnt, b_c2, batch3, s, W_fs, W_fg, b_f, W_o_pad,
              b_o_pad):
    _, n, d = agg.shape
    bn = 1000
    nb = n // bn
    bs = s.shape[0]

    def body(agg_ref, p_ref, cnt_ref, b_ref, bat_ref, s_ref, wfs_ref,
             wfg_ref, bf_ref, wo_ref, bo_ref, o_ref, gsum, cntv):
        i = pl.program_id(0)

        @pl.when(i == 0)
        def _():
            gsum[...] = jnp.zeros_like(gsum)
            cntv[...] = jnp.zeros_like(cntv)

        deg = cnt_ref[0, :, 0:1] + cnt_ref[1, :, 0:1] + 1.0
        dinv = lax.rsqrt(deg)
        h2 = jnp.maximum(
            dinv * (agg_ref[0] + agg_ref[1] + p_ref[...]) + b_ref[...], 0.0)
        bvals = bat_ref[0]
        biota = lax.broadcasted_iota(jnp.int32, (bs, bn), 0)
        mask = (biota == bvals).astype(jnp.float32)
        gsum[...] += jnp.dot(mask, h2, preferred_element_type=jnp.float32,
                             precision=_HIGH)
        cntv[...] += jnp.sum(mask, axis=1, keepdims=True)

        @pl.when(i == nb - 1)
        def _():
            g = gsum[...] / jnp.maximum(cntv[...], 1.0)
            fused = jnp.maximum(
                jnp.dot(s_ref[...], wfs_ref[...],
                        preferred_element_type=jnp.float32, precision=_HIGH)
                + jnp.dot(g, wfg_ref[...],
                          preferred_element_type=jnp.float32, precision=_HIGH)
                + bf_ref[...], 0.0)
            o_ref[...] = jnp.dot(fused, wo_ref[...],
                                 preferred_element_type=jnp.float32,
                                 precision=_HIGH) + bo_ref[...]

    h = W_fs.shape[1]
    return pl.pallas_call(
        body,
        grid=(nb,),
        in_specs=[
            pl.BlockSpec((NC, bn, d), lambda i: (0, i, 0)),
            pl.BlockSpec((bn, d), lambda i: (i, 0)),
            pl.BlockSpec((NC, bn, CW), lambda i: (0, i, 0)),
            pl.BlockSpec((1, d), lambda i: (0, 0)),
            pl.BlockSpec((1, 1, bn), lambda i: (i, 0, 0)),
            pl.BlockSpec((bs, h), lambda i: (0, 0)),
            pl.BlockSpec((h, h), lambda i: (0, 0)),
            pl.BlockSpec((h, h), lambda i: (0, 0)),
            pl.BlockSpec((1, h), lambda i: (0, 0)),
            pl.BlockSpec((h, h), lambda i: (0, 0)),
            pl.BlockSpec((1, h), lambda i: (0, 0)),
        ],
        out_specs=pl.BlockSpec((bs, h), lambda i: (0, 0)),
        out_shape=jax.ShapeDtypeStruct((bs, h), jnp.float32),
        scratch_shapes=[
            pltpu.VMEM((bs, h), jnp.float32),
            pltpu.VMEM((bs, 1), jnp.float32),
        ],
    )(agg, p2, cnt, b_c2, batch3, s, W_fs, W_fg, b_f, W_o_pad, b_o_pad)


def kernel(smiles_input, x, edge_index, batch, W_s1, b_s1, W_s2, b_s2,
           W_c1, b_c1, W_c2, b_c2, W_f, b_f, W_o, b_o):
    n, d = x.shape
    src = edge_index[0]
    dst = edge_index[1]
    h = W_s1.shape[1]
    c_out = W_o.shape[1]

    cnt = _sc_degree_counts(dst, n)
    p1, s = _tc_prep(x, W_c1, cnt, smiles_input,
                     W_s1, b_s1.reshape(1, -1), W_s2, b_s2.reshape(1, -1))
    agg1 = _sc_edge_agg(p1, src, dst)
    p2 = _tc_mid(agg1, p1, cnt, b_c1.reshape(1, -1), W_c2)
    agg2 = _sc_edge_agg(p2, src, dst)

    bn = 1000
    batch3 = batch.reshape(n // bn, 1, bn)
    W_o_pad = jnp.pad(W_o, ((0, 0), (0, h - c_out)))
    b_o_pad = jnp.pad(b_o, (0, h - c_out)).reshape(1, -1)
    outp = _tc_final(agg2, p2, cnt, b_c2.reshape(1, -1), batch3, s,
                     W_f[:h], W_f[h:], b_f.reshape(1, -1), W_o_pad, b_o_pad)
    return outp[:, :c_out]


# speedup vs baseline: 27.2760x; 1.1414x over previous
import functools

import jax
import jax.numpy as jnp
from jax import lax
from jax.experimental import pallas as pl
from jax.experimental.pallas import tpu as pltpu
from jax.experimental.pallas import tpu_sc as plsc

NC, NS = 2, 16
NW = NC * NS
CW = 16
EK = 80

def _sc_mesh():
    return plsc.VectorSubcoreMesh(
        core_axis_name="c", subcore_axis_name="s",
        num_cores=NC, num_subcores=NS)

_HIGH = lax.Precision.HIGHEST


CH = 200


def _sc_degree_counts(dst, n_nodes):
    e = dst.shape[0]
    per_w = e // NW
    nb = per_w // EK
    half = nb // 2
    n_ch = n_nodes // CH
    jmax = (n_ch + NS - 1) // NS

    @functools.partial(
        pl.kernel,
        out_type=jax.ShapeDtypeStruct((NC, n_nodes, CW), jnp.float32),
        mesh=_sc_mesh(),
        scratch_types=[
            pltpu.VMEM_SHARED((n_nodes, CW), jnp.float32),
            pltpu.VMEM((CH, CW), jnp.float32),
            pltpu.VMEM((EK, CW), jnp.float32),
            pltpu.VMEM((EK,), jnp.int32),
            pltpu.VMEM((EK,), jnp.int32),
            pltpu.SemaphoreType.DMA,
            pltpu.SemaphoreType.DMA,
        ],
    )
    def k(dst_hbm, out_hbm, acc, zbuf, ones, didx0, didx1, dsem0, dsem1):
        cid = lax.axis_index("c")
        sid = lax.axis_index("s")
        wid = sid * NC + cid
        zv = jnp.zeros((16,), jnp.float32)
        ov = jnp.ones((16,), jnp.float32)

        def fill_z(i, _):
            zbuf[i, pl.ds(0, 16)] = zv
            return 0

        lax.fori_loop(0, CH, fill_z, 0)

        def fill_o(i, _):
            ones[i, pl.ds(0, 16)] = ov
            return 0

        lax.fori_loop(0, EK, fill_o, 0)

        def zero_chunk(j, _):
            t = sid + j * NS

            @pl.when(t < n_ch)
            def _():
                pltpu.sync_copy(zbuf, acc.at[pl.ds(t * CH, CH)])

            return 0

        lax.fori_loop(0, jmax, zero_chunk, 0)
        plsc.subcore_barrier()

        base = wid * per_w

        def dsl(c):
            return dst_hbm.at[pl.ds(base + c * EK, EK)]

        def idrain(buf, sem):
            pltpu.make_async_copy(dst_hbm.at[pl.ds(0, EK)], buf, sem).wait()

        pltpu.async_copy(dsl(0), didx0, dsem0)
        pltpu.async_copy(dsl(1), didx1, dsem1)

        def pair(cc, _):
            c0 = 2 * cc
            idrain(didx0, dsem0)
            pltpu.sync_copy(ones, acc.at[didx0], add=True)
            pltpu.async_copy(dsl(c0 + 2), didx0, dsem0)
            idrain(didx1, dsem1)
            pltpu.sync_copy(ones, acc.at[didx1], add=True)

            @pl.when(cc < half - 1)
            def _():
                pltpu.async_copy(dsl(c0 + 3), didx1, dsem1)

            return 0

        lax.fori_loop(0, half, pair, 0)
        idrain(didx0, dsem0)
        pltpu.sync_copy(ones, acc.at[didx0], add=True)
        plsc.subcore_barrier()

        def wb(j, _):
            t = sid + j * NS

            @pl.when(t < n_ch)
            def _():
                sl = pl.ds(t * CH, CH)
                pltpu.sync_copy(acc.at[sl], zbuf)
                pltpu.sync_copy(zbuf, out_hbm.at[cid, sl])

            return 0

        lax.fori_loop(0, jmax, wb, 0)

    return k(dst)


def _sc_edge_agg(p, src, dst):
    n, d = p.shape
    e = src.shape[0]
    per_w = e // NW
    nb = per_w // EK
    half = nb // 2
    n_zc = n // EK
    jmax = (n_zc + NS - 1) // NS

    @functools.partial(
        pl.kernel,
        out_type=jax.ShapeDtypeStruct((NC, n, d), jnp.float32),
        mesh=_sc_mesh(),
        scratch_types=[
            pltpu.VMEM_SHARED((n, d), jnp.float32),
            pltpu.VMEM((EK,), jnp.int32),
            pltpu.VMEM((EK,), jnp.int32),
            pltpu.VMEM((EK,), jnp.int32),
            pltpu.VMEM((EK,), jnp.int32),
            pltpu.VMEM((EK, d), jnp.float32),
            pltpu.VMEM((EK, d), jnp.float32),
            pltpu.SemaphoreType.DMA,
            pltpu.SemaphoreType.DMA,
            pltpu.SemaphoreType.DMA,
            pltpu.SemaphoreType.DMA,
            pltpu.SemaphoreType.DMA,
            pltpu.SemaphoreType.DMA,
        ],
    )
    def k(p_hbm, src_hbm, dst_hbm, out_hbm, acc, sidx0, sidx1, didx0, didx1,
          rows0, rows1, g0, g1, si0, si1, di0, di1):
        cid = lax.axis_index("c")
        sid = lax.axis_index("s")
        wid = sid * NC + cid
        zv = jnp.zeros((16,), jnp.float32)

        def fill_z(i, _):
            def col(j, _):
                rows0[i, pl.ds(j * 16, 16)] = zv
                return 0

            lax.fori_loop(0, d // 16, col, 0)
            return 0

        lax.fori_loop(0, EK, fill_z, 0)

        def zero_chunk(j, _):
            t = sid + j * NS

            @pl.when(t < n_zc)
            def _():
                pltpu.sync_copy(rows0, acc.at[pl.ds(t * EK, EK)])

            return 0

        lax.fori_loop(0, jmax, zero_chunk, 0)
        plsc.subcore_barrier()

        base = wid * per_w

        def ssl(c):
            return src_hbm.at[pl.ds(base + c * EK, EK)]

        def dsl(c):
            return dst_hbm.at[pl.ds(base + c * EK, EK)]

        def idrain(buf, sem):
            pltpu.make_async_copy(src_hbm.at[pl.ds(0, EK)], buf, sem).wait()

        def gdrain(buf, sem):
            pltpu.make_async_copy(p_hbm.at[pl.ds(0, EK)], buf, sem).wait()

        pltpu.async_copy(ssl(0), sidx0, si0)
        pltpu.async_copy(dsl(0), didx0, di0)
        pltpu.async_copy(ssl(1), sidx1, si1)
        pltpu.async_copy(dsl(1), didx1, di1)
        idrain(sidx0, si0)
        pltpu.async_copy(p_hbm.at[sidx0], rows0, g0)
        idrain(sidx1, si1)
        pltpu.async_copy(p_hbm.at[sidx1], rows1, g1)

        def pair(cc, _):
            c0 = 2 * cc
            gdrain(rows0, g0)
            pltpu.async_copy(ssl(c0 + 2), sidx0, si0)
            idrain(didx0, di0)
            pltpu.sync_copy(rows0, acc.at[didx0], add=True)
            idrain(sidx0, si0)
            pltpu.async_copy(p_hbm.at[sidx0], rows0, g0)
            pltpu.async_copy(dsl(c0 + 2), didx0, di0)
            gdrain(rows1, g1)

            @pl.when(cc < half - 1)
            def _():
                pltpu.async_copy(ssl(c0 + 3), sidx1, si1)

            idrain(didx1, di1)
            pltpu.sync_copy(rows1, acc.at[didx1], add=True)

            @pl.when(cc < half - 1)
            def _():
                idrain(sidx1, si1)
                pltpu.async_copy(p_hbm.at[sidx1], rows1, g1)
                pltpu.async_copy(dsl(c0 + 3), didx1, di1)

            return 0

        lax.fori_loop(0, half, pair, 0)
        gdrain(rows0, g0)
        idrain(didx0, di0)
        pltpu.sync_copy(rows0, acc.at[didx0], add=True)
        plsc.subcore_barrier()

        def wb(j, _):
            t = sid + j * NS

            @pl.when(t < n_zc)
            def _():
                sl = pl.ds(t * EK, EK)
                pltpu.sync_copy(acc.at[sl], rows0)
                pltpu.sync_copy(rows0, out_hbm.at[cid, sl])

            return 0

        lax.fori_loop(0, jmax, wb, 0)

    return k(p, src, dst)


def _tc_prep(x, W_c1, cnt, smiles, W_s1, b_s1, W_s2, b_s2):
    n, d = x.shape
    bn = 1000
    nb = n // bn
    bs, ss = smiles.shape
    h = W_s1.shape[1]

    def body(x_ref, w_ref, cnt_ref, sm_ref, ws1_ref, bs1_ref, ws2_ref,
             bs2_ref, p_ref, s_ref):
        i = pl.program_id(0)
        deg = cnt_ref[0, :, 0:1] + cnt_ref[1, :, 0:1] + 1.0
        dinv = lax.rsqrt(deg)
        p_ref[...] = jnp.dot(x_ref[...], w_ref[...],
                             preferred_element_type=jnp.float32,
                             precision=_HIGH) * dinv

        @pl.when(i == 0)
        def _():
            t = jnp.maximum(
                jnp.dot(sm_ref[...], ws1_ref[...],
                        preferred_element_type=jnp.float32,
                        precision=_HIGH) + bs1_ref[...], 0.0)
            s_ref[...] = jnp.dot(t, ws2_ref[...],
                                 preferred_element_type=jnp.float32,
                                 precision=_HIGH) + bs2_ref[...]

    return pl.pallas_call(
        body,
        grid=(nb,),
        in_specs=[
            pl.BlockSpec((bn, d), lambda i: (i, 0)),
            pl.BlockSpec((d, d), lambda i: (0, 0)),
            pl.BlockSpec((NC, bn, CW), lambda i: (0, i, 0)),
            pl.BlockSpec((bs, ss), lambda i: (0, 0)),
            pl.BlockSpec((ss, h), lambda i: (0, 0)),
            pl.BlockSpec((1, h), lambda i: (0, 0)),
            pl.BlockSpec((h, h), lambda i: (0, 0)),
            pl.BlockSpec((1, h), lambda i: (0, 0)),
        ],
        out_specs=[
            pl.BlockSpec((bn, d), lambda i: (i, 0)),
            pl.BlockSpec((bs, h), lambda i: (0, 0)),
        ],
        out_shape=[
            jax.ShapeDtypeStruct((n, d), jnp.float32),
            jax.ShapeDtypeStruct((bs, h), jnp.float32),
        ],
    )(x, W_c1, cnt, smiles, W_s1, b_s1, W_s2, b_s2)


def _tc_mid(agg, p1, cnt, b_c1, W_c2):
    _, n, d = agg.shape
    bn = 1000
    nb = n // bn

    def body(agg_ref, p_ref, cnt_ref, b_ref, w_ref, o_ref):
        deg = cnt_ref[0, :, 0:1] + cnt_ref[1, :, 0:1] + 1.0
        dinv = lax.rsqrt(deg)
        h1 = jnp.maximum(
            dinv * (agg_ref[0] + agg_ref[1] + p_ref[...]) + b_ref[...], 0.0)
        o_ref[...] = jnp.dot(h1, w_ref[...],
                             preferred_element_type=jnp.float32,
                             precision=_HIGH) * dinv

    return pl.pallas_call(
        body,
        grid=(nb,),
        in_specs=[
            pl.BlockSpec((NC, bn, d), lambda i: (0, i, 0)),
            pl.BlockSpec((bn, d), lambda i: (i, 0)),
            pl.BlockSpec((NC, bn, CW), lambda i: (0, i, 0)),
            pl.BlockSpec((1, d), lambda i: (0, 0)),
            pl.BlockSpec((d, d), lambda i: (0, 0)),
        ],
        out_specs=pl.BlockSpec((bn, d), lambda i: (i, 0)),
        out_shape=jax.ShapeDtypeStruct((n, d), jnp.float32),
    )(agg, p1, cnt, b_c1, W_c2)


def _tc_final(agg, p2, cnt, b_c2, batch3, s, W_fs, W_fg, b_f, W_o_pad,
              b_o_pad):
    _, n, d = agg.shape
    bn = 1000
    nb = n // bn
    bs = s.shape[0]

    def body(agg_ref, p_ref, cnt_ref, b_ref, bat_ref, s_ref, wfs_ref,
             wfg_ref, bf_ref, wo_ref, bo_ref, o_ref, gsum, cntv):
        i = pl.program_id(0)

        @pl.when(i == 0)
        def _():
            gsum[...] = jnp.zeros_like(gsum)
            cntv[...] = jnp.zeros_like(cntv)

        deg = cnt_ref[0, :, 0:1] + cnt_ref[1, :, 0:1] + 1.0
        dinv = lax.rsqrt(deg)
        h2 = jnp.maximum(
            dinv * (agg_ref[0] + agg_ref[1] + p_ref[...]) + b_ref[...], 0.0)
        bvals = bat_ref[0]
        biota = lax.broadcasted_iota(jnp.int32, (bs, bn), 0)
        mask = (biota == bvals).astype(jnp.float32)
        gsum[...] += jnp.dot(mask, h2, preferred_element_type=jnp.float32,
                             precision=_HIGH)
        cntv[...] += jnp.sum(mask, axis=1, keepdims=True)

        @pl.when(i == nb - 1)
        def _():
            g = gsum[...] / jnp.maximum(cntv[...], 1.0)
            fused = jnp.maximum(
                jnp.dot(s_ref[...], wfs_ref[...],
                        preferred_element_type=jnp.float32, precision=_HIGH)
                + jnp.dot(g, wfg_ref[...],
                          preferred_element_type=jnp.float32, precision=_HIGH)
                + bf_ref[...], 0.0)
            o_ref[...] = jnp.dot(fused, wo_ref[...],
                                 preferred_element_type=jnp.float32,
                                 precision=_HIGH) + bo_ref[...]

    h = W_fs.shape[1]
    return pl.pallas_call(
        body,
        grid=(nb,),
        in_specs=[
            pl.BlockSpec((NC, bn, d), lambda i: (0, i, 0)),
            pl.BlockSpec((bn, d), lambda i: (i, 0)),
            pl.BlockSpec((NC, bn, CW), lambda i: (0, i, 0)),
            pl.BlockSpec((1, d), lambda i: (0, 0)),
            pl.BlockSpec((1, 1, bn), lambda i: (i, 0, 0)),
            pl.BlockSpec((bs, h), lambda i: (0, 0)),
            pl.BlockSpec((h, h), lambda i: (0, 0)),
            pl.BlockSpec((h, h), lambda i: (0, 0)),
            pl.BlockSpec((1, h), lambda i: (0, 0)),
            pl.BlockSpec((h, h), lambda i: (0, 0)),
            pl.BlockSpec((1, h), lambda i: (0, 0)),
        ],
        out_specs=pl.BlockSpec((bs, h), lambda i: (0, 0)),
        out_shape=jax.ShapeDtypeStruct((bs, h), jnp.float32),
        scratch_shapes=[
            pltpu.VMEM((bs, h), jnp.float32),
            pltpu.VMEM((bs, 1), jnp.float32),
        ],
    )(agg, p2, cnt, b_c2, batch3, s, W_fs, W_fg, b_f, W_o_pad, b_o_pad)


def kernel(smiles_input, x, edge_index, batch, W_s1, b_s1, W_s2, b_s2,
           W_c1, b_c1, W_c2, b_c2, W_f, b_f, W_o, b_o):
    n, d = x.shape
    src = edge_index[0]
    dst = edge_index[1]
    h = W_s1.shape[1]
    c_out = W_o.shape[1]

    cnt = _sc_degree_counts(dst, n)
    p1, s = _tc_prep(x, W_c1, cnt, smiles_input,
                     W_s1, b_s1.reshape(1, -1), W_s2, b_s2.reshape(1, -1))
    agg1 = _sc_edge_agg(p1, src, dst)
    p2 = _tc_mid(agg1, p1, cnt, b_c1.reshape(1, -1), W_c2)
    agg2 = _sc_edge_agg(p2, src, dst)

    bn = 1000
    batch3 = batch.reshape(n // bn, 1, bn)
    W_o_pad = jnp.pad(W_o, ((0, 0), (0, h - c_out)))
    b_o_pad = jnp.pad(b_o, (0, h - c_out)).reshape(1, -1)
    outp = _tc_final(agg2, p2, cnt, b_c2.reshape(1, -1), batch3, s,
                     W_f[:h], W_f[h:], b_f.reshape(1, -1), W_o_pad, b_o_pad)
    return outp[:, :c_out]


# speedup vs baseline: 27.5859x; 1.0114x over previous
import functools

import jax
import jax.numpy as jnp
from jax import lax
from jax.experimental import pallas as pl
from jax.experimental.pallas import tpu as pltpu
from jax.experimental.pallas import tpu_sc as plsc

NC, NS = 2, 16
NW = NC * NS
CW = 16
EK = 80

def _sc_mesh():
    return plsc.VectorSubcoreMesh(
        core_axis_name="c", subcore_axis_name="s",
        num_cores=NC, num_subcores=NS)

_HIGH = lax.Precision.HIGHEST


CH = 200


def _sc_degree_counts(dst, n_nodes):
    e = dst.shape[0]
    per_w = e // NW
    nb = per_w // EK
    half = nb // 2
    n_ch = n_nodes // CH
    jmax = (n_ch + NS - 1) // NS

    @functools.partial(
        pl.kernel,
        out_type=jax.ShapeDtypeStruct((NC, n_nodes, CW), jnp.float32),
        mesh=_sc_mesh(),
        scratch_types=[
            pltpu.VMEM_SHARED((n_nodes, CW), jnp.float32),
            pltpu.VMEM((CH, CW), jnp.float32),
            pltpu.VMEM((EK, CW), jnp.float32),
            pltpu.VMEM((EK,), jnp.int32),
            pltpu.VMEM((EK,), jnp.int32),
            pltpu.SemaphoreType.DMA,
            pltpu.SemaphoreType.DMA,
        ],
    )
    def k(dst_hbm, out_hbm, acc, zbuf, ones, didx0, didx1, dsem0, dsem1):
        cid = lax.axis_index("c")
        sid = lax.axis_index("s")
        wid = sid * NC + cid
        zv = jnp.zeros((16,), jnp.float32)
        ov = jnp.ones((16,), jnp.float32)

        def fill_z(i, _):
            zbuf[i, pl.ds(0, 16)] = zv
            return 0

        lax.fori_loop(0, CH, fill_z, 0)

        def fill_o(i, _):
            ones[i, pl.ds(0, 16)] = ov
            return 0

        lax.fori_loop(0, EK, fill_o, 0)

        def zero_chunk(j, _):
            t = sid + j * NS

            @pl.when(t < n_ch)
            def _():
                pltpu.sync_copy(zbuf, acc.at[pl.ds(t * CH, CH)])

            return 0

        lax.fori_loop(0, jmax, zero_chunk, 0)
        plsc.subcore_barrier()

        base = wid * per_w

        def dsl(c):
            return dst_hbm.at[pl.ds(base + c * EK, EK)]

        def idrain(buf, sem):
            pltpu.make_async_copy(dst_hbm.at[pl.ds(0, EK)], buf, sem).wait()

        pltpu.async_copy(dsl(0), didx0, dsem0)
        pltpu.async_copy(dsl(1), didx1, dsem1)

        def pair(cc, _):
            c0 = 2 * cc
            idrain(didx0, dsem0)
            pltpu.sync_copy(ones, acc.at[didx0], add=True)
            pltpu.async_copy(dsl(c0 + 2), didx0, dsem0)
            idrain(didx1, dsem1)
            pltpu.sync_copy(ones, acc.at[didx1], add=True)

            @pl.when(cc < half - 1)
            def _():
                pltpu.async_copy(dsl(c0 + 3), didx1, dsem1)

            return 0

        lax.fori_loop(0, half, pair, 0)
        idrain(didx0, dsem0)
        pltpu.sync_copy(ones, acc.at[didx0], add=True)
        plsc.subcore_barrier()

        def wb(j, _):
            t = sid + j * NS

            @pl.when(t < n_ch)
            def _():
                sl = pl.ds(t * CH, CH)
                pltpu.sync_copy(acc.at[sl], zbuf)
                pltpu.sync_copy(zbuf, out_hbm.at[cid, sl])

            return 0

        lax.fori_loop(0, jmax, wb, 0)

    return k(dst)


def _sc_edge_agg(p, src, dst):
    n, d = p.shape
    e = src.shape[0]
    per_w = e // NW
    nb = per_w // EK
    half = nb // 2
    n_zc = n // EK
    jmax = (n_zc + NS - 1) // NS

    @functools.partial(
        pl.kernel,
        out_type=jax.ShapeDtypeStruct((NC, n, d), jnp.float32),
        mesh=_sc_mesh(),
        scratch_types=[
            pltpu.VMEM_SHARED((n, d), jnp.float32),
            pltpu.VMEM((EK,), jnp.int32),
            pltpu.VMEM((EK,), jnp.int32),
            pltpu.VMEM((EK,), jnp.int32),
            pltpu.VMEM((EK,), jnp.int32),
            pltpu.VMEM((EK, d), jnp.float32),
            pltpu.VMEM((EK, d), jnp.float32),
            pltpu.SemaphoreType.DMA,
            pltpu.SemaphoreType.DMA,
            pltpu.SemaphoreType.DMA,
            pltpu.SemaphoreType.DMA,
            pltpu.SemaphoreType.DMA,
            pltpu.SemaphoreType.DMA,
        ],
    )
    def k(p_hbm, src_hbm, dst_hbm, out_hbm, acc, sidx0, sidx1, didx0, didx1,
          rows0, rows1, g0, g1, si0, si1, di0, di1):
        cid = lax.axis_index("c")
        sid = lax.axis_index("s")
        wid = sid * NC + cid
        zv = jnp.zeros((16,), jnp.float32)

        def fill_z(i, _):
            def col(j, _):
                rows0[i, pl.ds(j * 16, 16)] = zv
                return 0

            lax.fori_loop(0, d // 16, col, 0)
            return 0

        lax.fori_loop(0, EK, fill_z, 0)

        def zero_chunk(j, _):
            t = sid + j * NS

            @pl.when(t < n_zc)
            def _():
                pltpu.sync_copy(rows0, acc.at[pl.ds(t * EK, EK)])

            return 0

        lax.fori_loop(0, jmax, zero_chunk, 0)
        plsc.subcore_barrier()

        base = wid * per_w

        def ssl(c):
            return src_hbm.at[pl.ds(base + c * EK, EK)]

        def dsl(c):
            return dst_hbm.at[pl.ds(base + c * EK, EK)]

        def idrain(buf, sem):
            pltpu.make_async_copy(src_hbm.at[pl.ds(0, EK)], buf, sem).wait()

        def gdrain(buf, sem):
            pltpu.make_async_copy(p_hbm.at[pl.ds(0, EK)], buf, sem).wait()

        pltpu.async_copy(ssl(0), sidx0, si0)
        pltpu.async_copy(dsl(0), didx0, di0)
        pltpu.async_copy(ssl(1), sidx1, si1)
        pltpu.async_copy(dsl(1), didx1, di1)
        idrain(sidx0, si0)
        pltpu.async_copy(p_hbm.at[sidx0], rows0, g0)
        idrain(sidx1, si1)
        pltpu.async_copy(p_hbm.at[sidx1], rows1, g1)

        def pair(cc, _):
            c0 = 2 * cc
            gdrain(rows0, g0)
            pltpu.async_copy(ssl(c0 + 2), sidx0, si0)
            idrain(didx0, di0)
            pltpu.sync_copy(rows0, acc.at[didx0], add=True)
            idrain(sidx0, si0)
            pltpu.async_copy(p_hbm.at[sidx0], rows0, g0)
            pltpu.async_copy(dsl(c0 + 2), didx0, di0)
            gdrain(rows1, g1)

            @pl.when(cc < half - 1)
            def _():
                pltpu.async_copy(ssl(c0 + 3), sidx1, si1)

            idrain(didx1, di1)
            pltpu.sync_copy(rows1, acc.at[didx1], add=True)

            @pl.when(cc < half - 1)
            def _():
                idrain(sidx1, si1)
                pltpu.async_copy(p_hbm.at[sidx1], rows1, g1)
                pltpu.async_copy(dsl(c0 + 3), didx1, di1)

            return 0

        lax.fori_loop(0, half, pair, 0)
        gdrain(rows0, g0)
        idrain(didx0, di0)
        pltpu.sync_copy(rows0, acc.at[didx0], add=True)
        plsc.subcore_barrier()

        def wb(j, _):
            t = sid + j * NS

            @pl.when(t < n_zc)
            def _():
                sl = pl.ds(t * EK, EK)
                pltpu.sync_copy(acc.at[sl], rows0)
                pltpu.sync_copy(rows0, out_hbm.at[cid, sl])

            return 0

        lax.fori_loop(0, jmax, wb, 0)

    return k(p, src, dst)


def _tc_prep(x, W_c1, smiles, W_s1, b_s1, W_s2, b_s2):
    n, d = x.shape
    bn = 1000
    nb = n // bn
    bs, ss = smiles.shape
    h = W_s1.shape[1]

    def body(x_ref, w_ref, sm_ref, ws1_ref, bs1_ref, ws2_ref,
             bs2_ref, q_ref, s_ref):
        i = pl.program_id(0)
        q_ref[...] = jnp.dot(x_ref[...], w_ref[...],
                             preferred_element_type=jnp.float32,
                             precision=_HIGH)

        @pl.when(i == 0)
        def _():
            t = jnp.maximum(
                jnp.dot(sm_ref[...], ws1_ref[...],
                        preferred_element_type=jnp.float32,
                        precision=_HIGH) + bs1_ref[...], 0.0)
            s_ref[...] = jnp.dot(t, ws2_ref[...],
                                 preferred_element_type=jnp.float32,
                                 precision=_HIGH) + bs2_ref[...]

    return pl.pallas_call(
        body,
        grid=(nb,),
        in_specs=[
            pl.BlockSpec((bn, d), lambda i: (i, 0)),
            pl.BlockSpec((d, d), lambda i: (0, 0)),
            pl.BlockSpec((bs, ss), lambda i: (0, 0)),
            pl.BlockSpec((ss, h), lambda i: (0, 0)),
            pl.BlockSpec((1, h), lambda i: (0, 0)),
            pl.BlockSpec((h, h), lambda i: (0, 0)),
            pl.BlockSpec((1, h), lambda i: (0, 0)),
        ],
        out_specs=[
            pl.BlockSpec((bn, d), lambda i: (i, 0)),
            pl.BlockSpec((bs, h), lambda i: (0, 0)),
        ],
        out_shape=[
            jax.ShapeDtypeStruct((n, d), jnp.float32),
            jax.ShapeDtypeStruct((bs, h), jnp.float32),
        ],
    )(x, W_c1, smiles, W_s1, b_s1, W_s2, b_s2)


def _tc_scale(q, cnt):
    n, d = q.shape
    bn = 1000
    nb = n // bn

    def body(q_ref, cnt_ref, p_ref):
        deg = cnt_ref[0, :, 0:1] + cnt_ref[1, :, 0:1] + 1.0
        p_ref[...] = q_ref[...] * lax.rsqrt(deg)

    return pl.pallas_call(
        body,
        grid=(nb,),
        in_specs=[
            pl.BlockSpec((bn, d), lambda i: (i, 0)),
            pl.BlockSpec((NC, bn, CW), lambda i: (0, i, 0)),
        ],
        out_specs=pl.BlockSpec((bn, d), lambda i: (i, 0)),
        out_shape=jax.ShapeDtypeStruct((n, d), jnp.float32),
    )(q, cnt)


def _tc_mid(agg, p1, cnt, b_c1, W_c2):
    _, n, d = agg.shape
    bn = 1000
    nb = n // bn

    def body(agg_ref, p_ref, cnt_ref, b_ref, w_ref, o_ref):
        deg = cnt_ref[0, :, 0:1] + cnt_ref[1, :, 0:1] + 1.0
        dinv = lax.rsqrt(deg)
        h1 = jnp.maximum(
            dinv * (agg_ref[0] + agg_ref[1] + p_ref[...]) + b_ref[...], 0.0)
        o_ref[...] = jnp.dot(h1, w_ref[...],
                             preferred_element_type=jnp.float32,
                             precision=_HIGH) * dinv

    return pl.pallas_call(
        body,
        grid=(nb,),
        in_specs=[
            pl.BlockSpec((NC, bn, d), lambda i: (0, i, 0)),
            pl.BlockSpec((bn, d), lambda i: (i, 0)),
            pl.BlockSpec((NC, bn, CW), lambda i: (0, i, 0)),
            pl.BlockSpec((1, d), lambda i: (0, 0)),
            pl.BlockSpec((d, d), lambda i: (0, 0)),
        ],
        out_specs=pl.BlockSpec((bn, d), lambda i: (i, 0)),
        out_shape=jax.ShapeDtypeStruct((n, d), jnp.float32),
    )(agg, p1, cnt, b_c1, W_c2)


def _tc_final(agg, p2, cnt, b_c2, batch3, s, W_fs, W_fg, b_f, W_o_pad,
              b_o_pad):
    _, n, d = agg.shape
    bn = 1000
    nb = n // bn
    bs = s.shape[0]

    def body(agg_ref, p_ref, cnt_ref, b_ref, bat_ref, s_ref, wfs_ref,
             wfg_ref, bf_ref, wo_ref, bo_ref, o_ref, gsum, cntv):
        i = pl.program_id(0)

        @pl.when(i == 0)
        def _():
            gsum[...] = jnp.zeros_like(gsum)
            cntv[...] = jnp.zeros_like(cntv)

        deg = cnt_ref[0, :, 0:1] + cnt_ref[1, :, 0:1] + 1.0
        dinv = lax.rsqrt(deg)
        h2 = jnp.maximum(
            dinv * (agg_ref[0] + agg_ref[1] + p_ref[...]) + b_ref[...], 0.0)
        bvals = bat_ref[0]
        biota = lax.broadcasted_iota(jnp.int32, (bs, bn), 0)
        mask = (biota == bvals).astype(jnp.float32)
        gsum[...] += jnp.dot(mask, h2, preferred_element_type=jnp.float32,
                             precision=_HIGH)
        cntv[...] += jnp.sum(mask, axis=1, keepdims=True)

        @pl.when(i == nb - 1)
        def _():
            g = gsum[...] / jnp.maximum(cntv[...], 1.0)
            fused = jnp.maximum(
                jnp.dot(s_ref[...], wfs_ref[...],
                        preferred_element_type=jnp.float32, precision=_HIGH)
                + jnp.dot(g, wfg_ref[...],
                          preferred_element_type=jnp.float32, precision=_HIGH)
                + bf_ref[...], 0.0)
            o_ref[...] = jnp.dot(fused, wo_ref[...],
                                 preferred_element_type=jnp.float32,
                                 precision=_HIGH) + bo_ref[...]

    h = W_fs.shape[1]
    return pl.pallas_call(
        body,
        grid=(nb,),
        in_specs=[
            pl.BlockSpec((NC, bn, d), lambda i: (0, i, 0)),
            pl.BlockSpec((bn, d), lambda i: (i, 0)),
            pl.BlockSpec((NC, bn, CW), lambda i: (0, i, 0)),
            pl.BlockSpec((1, d), lambda i: (0, 0)),
            pl.BlockSpec((1, 1, bn), lambda i: (i, 0, 0)),
            pl.BlockSpec((bs, h), lambda i: (0, 0)),
            pl.BlockSpec((h, h), lambda i: (0, 0)),
            pl.BlockSpec((h, h), lambda i: (0, 0)),
            pl.BlockSpec((1, h), lambda i: (0, 0)),
            pl.BlockSpec((h, h), lambda i: (0, 0)),
            pl.BlockSpec((1, h), lambda i: (0, 0)),
        ],
        out_specs=pl.BlockSpec((bs, h), lambda i: (0, 0)),
        out_shape=jax.ShapeDtypeStruct((bs, h), jnp.float32),
        scratch_shapes=[
            pltpu.VMEM((bs, h), jnp.float32),
            pltpu.VMEM((bs, 1), jnp.float32),
        ],
    )(agg, p2, cnt, b_c2, batch3, s, W_fs, W_fg, b_f, W_o_pad, b_o_pad)


def kernel(smiles_input, x, edge_index, batch, W_s1, b_s1, W_s2, b_s2,
           W_c1, b_c1, W_c2, b_c2, W_f, b_f, W_o, b_o):
    n, d = x.shape
    src = edge_index[0]
    dst = edge_index[1]
    h = W_s1.shape[1]
    c_out = W_o.shape[1]

    q, s = _tc_prep(x, W_c1, smiles_input,
                    W_s1, b_s1.reshape(1, -1), W_s2, b_s2.reshape(1, -1))
    cnt = _sc_degree_counts(dst, n)
    p1 = _tc_scale(q, cnt)
    agg1 = _sc_edge_agg(p1, src, dst)
    p2 = _tc_mid(agg1, p1, cnt, b_c1.reshape(1, -1), W_c2)
    agg2 = _sc_edge_agg(p2, src, dst)

    bn = 1000
    batch3 = batch.reshape(n // bn, 1, bn)
    W_o_pad = jnp.pad(W_o, ((0, 0), (0, h - c_out)))
    b_o_pad = jnp.pad(b_o, (0, h - c_out)).reshape(1, -1)
    outp = _tc_final(agg2, p2, cnt, b_c2.reshape(1, -1), batch3, s,
                     W_f[:h], W_f[h:], b_f.reshape(1, -1), W_o_pad, b_o_pad)
    return outp[:, :c_out]
